# baseline (speedup 1.0000x reference)
import jax
import jax.numpy as jnp
from jax.experimental import pallas as pl
from jax.experimental.pallas import tpu as pltpu


def _body(k_hbm, x_ref):
    x_ref[...] = jnp.full_like(x_ref, 1.0)


def kernel(q, k):
    bsz, seq, d = k.shape
    k2 = k.reshape(bsz, seq // 2, d * 2)
    x = pl.pallas_call(
        _body,
        in_specs=[pl.BlockSpec(memory_space=pltpu.MemorySpace.HBM)],
        out_specs=pl.BlockSpec(memory_space=pltpu.MemorySpace.VMEM),
        out_shape=jax.ShapeDtypeStruct((8, 128), jnp.float32),
    )(k2)
    return jnp.sum(x) + jnp.sum(q[0, 0]) > 0


# speedup vs baseline: 3.3401x; 3.3401x over previous
import functools
import math

import jax
import jax.numpy as jnp
from jax.experimental import pallas as pl

_R = 256
_ROWS_PER_STEP = 8


def _select_body(x_ref, g_ref, o_ref, *, scale):
    xs = x_ref[...] * scale
    m = jnp.max(xs, axis=-1, keepdims=True)
    e = jnp.exp(xs - m)
    s = jnp.sum(e, axis=-1, keepdims=True)
    v = jnp.log(e / s + 1e-20) + g_ref[...]
    b = jax.lax.bitcast_convert_type(v, jnp.int32)
    key = b ^ ((b >> 31) & jnp.int32(0x7FFFFFFF))

    def body(i, t):
        cand = t ^ (jnp.int32(1) << (jnp.int32(31) - i))
        cnt = jnp.sum((key >= cand).astype(jnp.int32), axis=-1, keepdims=True)
        return jnp.where(cnt >= _R, cand, t)

    t0 = jnp.full((_ROWS_PER_STEP, 1), jnp.int32(-2147483648))
    t = jax.lax.fori_loop(0, 32, body, t0)
    o_ref[...] = key >= t


def kernel(q, k):
    d = q.shape[-1]
    bsz, seq, _ = k.shape
    a = jnp.matmul(q[..., :1, :], jnp.swapaxes(k[..., 1:, :], -2, -1))
    xs = a[:, 0, :]
    g = jax.random.gumbel(jax.random.key(42), xs.shape, xs.dtype)
    mask = pl.pallas_call(
        functools.partial(_select_body, scale=1.0 / math.sqrt(d)),
        grid=(bsz // _ROWS_PER_STEP,),
        in_specs=[
            pl.BlockSpec((_ROWS_PER_STEP, seq - 1), lambda i: (i, 0)),
            pl.BlockSpec((_ROWS_PER_STEP, seq - 1), lambda i: (i, 0)),
        ],
        out_specs=pl.BlockSpec((_ROWS_PER_STEP, seq - 1), lambda i: (i, 0)),
        out_shape=jax.ShapeDtypeStruct((bsz, seq - 1), jnp.bool_),
    )(xs, g)
    return jnp.concatenate([jnp.ones((bsz, 1), jnp.bool_), mask], axis=1)
